# baseline (speedup 1.0000x reference)
import jax
import jax.numpy as jnp
import numpy as np
from jax.experimental import pallas as pl

_F32 = jnp.float32
_BIAS = float(np.log(np.expm1(1.0)))


def _adj(e, n_src, n_dst):
    src = e[0, :]
    dst = e[1, :]
    n_e = src.shape[0]
    s_oh = (src[:, None] == jax.lax.broadcasted_iota(jnp.int32, (n_e, n_src), 1)
            ).astype(_F32)
    d_oh = (dst[:, None] == jax.lax.broadcasted_iota(jnp.int32, (n_e, n_dst), 1)
            ).astype(_F32)
    return jax.lax.dot_general(
        d_oh, s_oh, (((0,), (0,)), ((), ())), preferred_element_type=_F32)


def _mm(a, b):
    return jax.lax.dot_general(
        a, b, (((1,), (0,)), ((), ())), preferred_element_type=_F32)


def _body(x_joint, x_torso, Wcat, bcat, Wout, bout, ei_cat,
          loc_ref, scale_ref):
    W = Wcat[...]
    b = bcat[...]
    h_j = _mm(x_joint[...], W[768:770, :]) + b[0][None, :]
    h_t = _mm(x_torso[...], W[770:781, :]) + b[1][None, :]

    e = ei_cat[...]
    A_tj = _adj(e[:, 0:80], 10, 80)
    A_jt = _adj(e[:, 80:160], 80, 10)
    A_jj = _adj(e[:, 160:280], 80, 80)

    x1j = jnp.concatenate([_mm(A_tj, h_t), _mm(A_jj, h_j), h_j], axis=1)
    w1j = jnp.concatenate([W[0:128, :], W[128:192, :] + W[192:256, :]], axis=0)
    j1 = jnp.tanh(_mm(x1j, w1j) + (b[2] + b[3])[None, :])
    x1t = jnp.concatenate([_mm(A_jt, h_j), h_t], axis=1)
    t1 = jnp.tanh(_mm(x1t, W[256:384, :]) + b[4][None, :])

    x2j = jnp.concatenate([_mm(A_tj, t1), _mm(A_jj, j1), j1], axis=1)
    w2j = jnp.concatenate([W[384:512, :], W[512:576, :] + W[576:640, :]],
                          axis=0)
    j2 = jnp.tanh(_mm(x2j, w2j) + (b[5] + b[6])[None, :])

    Wbig = jnp.concatenate([Wout[h] for h in range(8)], axis=1)
    bbig = jnp.concatenate([bout[h] for h in range(8)], axis=0)
    out16 = _mm(j2, Wbig) + bbig[None, :]
    col = jax.lax.broadcasted_iota(jnp.int32, (80, 16), 1)
    head2 = 2 * (jax.lax.broadcasted_iota(jnp.int32, (80, 16), 0) % 8)
    loc = jnp.sum(jnp.where(col == head2, out16, 0.0), axis=1)
    pre = jnp.sum(jnp.where(col == head2 + 1, out16, 0.0), axis=1)
    scale = jnp.maximum(jax.nn.softplus(pre + _BIAS), 1e-4)
    loc_ref[...] = loc.reshape(10, 8)
    scale_ref[...] = scale.reshape(10, 8)


def kernel(x_joint, x_torso, Wj, bj, Wt, bt,
           W1_tj_rel, b1_tj, W1_tj_root, W1_jj_rel, b1_jj, W1_jj_root,
           W1_jt_rel, b1_jt, W1_jt_root,
           W2_tj_rel, b2_tj, W2_tj_root, W2_jj_rel, b2_jj, W2_jj_root,
           W2_jt_rel, b2_jt, W2_jt_root,
           Wout, bout, ei_tj, ei_jt, ei_jj):
    Wcat = jnp.concatenate(
        [W1_tj_rel, W1_jj_rel, W1_tj_root, W1_jj_root, W1_jt_rel, W1_jt_root,
         W2_tj_rel, W2_jj_rel, W2_tj_root, W2_jj_root, W2_jt_rel, W2_jt_root,
         Wj, Wt], axis=0)
    bcat = jnp.stack([bj, bt, b1_tj, b1_jj, b1_jt, b2_tj, b2_jj])
    ei_cat = jnp.concatenate(
        [ei_tj.astype(jnp.int32), ei_jt.astype(jnp.int32),
         ei_jj.astype(jnp.int32)], axis=1)
    loc, scale = pl.pallas_call(
        _body,
        out_shape=(jax.ShapeDtypeStruct((10, 8), _F32),
                   jax.ShapeDtypeStruct((10, 8), _F32)),
    )(x_joint, x_torso, Wcat, bcat, Wout, bout, ei_cat)
    return (loc, scale)


# speedup vs baseline: 2.4743x; 2.4743x over previous
import jax
import jax.numpy as jnp
import numpy as np
from jax.experimental import pallas as pl
from jax.experimental.pallas import tpu as pltpu

_F32 = jnp.float32
_BIAS = float(np.log(np.expm1(1.0)))

_ANY = pltpu.MemorySpace.HBM


def _adj(e, n_src, n_dst):
    src = e[0, :]
    dst = e[1, :]
    n_e = src.shape[0]
    s_oh = (src[:, None] == jax.lax.broadcasted_iota(jnp.int32, (n_e, n_src), 1)
            ).astype(_F32)
    d_oh = (dst[:, None] == jax.lax.broadcasted_iota(jnp.int32, (n_e, n_dst), 1)
            ).astype(_F32)
    return jax.lax.dot_general(
        d_oh, s_oh, (((0,), (0,)), ((), ())), preferred_element_type=_F32)


def _mm(a, b):
    return jax.lax.dot_general(
        a, b, (((1,), (0,)), ((), ())), preferred_element_type=_F32)


_SHAPES = [
    ((80, 2), _F32), ((10, 11), _F32),
    ((2, 64), _F32), ((64,), _F32),
    ((11, 64), _F32), ((64,), _F32),
    ((64, 64), _F32), ((64,), _F32), ((64, 64), _F32),
    ((64, 64), _F32), ((64,), _F32), ((64, 64), _F32),
    ((64, 64), _F32), ((64,), _F32), ((64, 64), _F32),
    ((64, 64), _F32), ((64,), _F32), ((64, 64), _F32),
    ((64, 64), _F32), ((64,), _F32), ((64, 64), _F32),
    None, None, None,
    ((8, 64, 2), _F32), ((8, 2), _F32),
    ((2, 80), jnp.int32), ((2, 80), jnp.int32), ((2, 120), jnp.int32),
]
_LIVE = [i for i, s in enumerate(_SHAPES) if s is not None]
_N_IN = len(_SHAPES)


def _body(*refs):
    hbm = refs[:_N_IN]
    loc_ref, scale_ref = refs[_N_IN], refs[_N_IN + 1]
    vmem = list(refs[_N_IN + 2:_N_IN + 2 + len(_LIVE)])
    sems = refs[_N_IN + 2 + len(_LIVE)]

    copies = {}
    for k, i in enumerate(_LIVE):
        copies[i] = pltpu.make_async_copy(hbm[i], vmem[k], sems.at[k])
        copies[i].start()
    v = {i: vmem[k] for k, i in enumerate(_LIVE)}

    def get(i):
        copies[i].wait()
        return v[i][...]

    A_tj = _adj(get(26), 10, 80)
    A_jt = _adj(get(27), 80, 10)
    A_jj = _adj(get(28), 80, 80)

    h_j = _mm(get(0), get(2)) + get(3)[None, :]
    h_t = _mm(get(1), get(4)) + get(5)[None, :]

    x1j = jnp.concatenate([_mm(A_tj, h_t), _mm(A_jj, h_j), h_j], axis=1)
    w1j = jnp.concatenate([get(6), get(9), get(8) + get(11)], axis=0)
    j1 = jnp.tanh(_mm(x1j, w1j) + (get(7) + get(10))[None, :])
    x1t = jnp.concatenate([_mm(A_jt, h_j), h_t], axis=1)
    w1t = jnp.concatenate([get(12), get(14)], axis=0)
    t1 = jnp.tanh(_mm(x1t, w1t) + get(13)[None, :])

    x2j = jnp.concatenate([_mm(A_tj, t1), _mm(A_jj, j1), j1], axis=1)
    w2j = jnp.concatenate([get(15), get(18), get(17) + get(20)], axis=0)
    j2 = jnp.tanh(_mm(x2j, w2j) + (get(16) + get(19))[None, :])

    Wout = get(24)
    bout = get(25)
    Wbig = jnp.concatenate([Wout[h] for h in range(8)], axis=1)
    bbig = jnp.concatenate([bout[h] for h in range(8)], axis=0)
    out16 = _mm(j2, Wbig) + bbig[None, :]
    col = jax.lax.broadcasted_iota(jnp.int32, (80, 16), 1)
    head2 = 2 * (jax.lax.broadcasted_iota(jnp.int32, (80, 16), 0) % 8)
    loc = jnp.sum(jnp.where(col == head2, out16, 0.0), axis=1)
    pre = jnp.sum(jnp.where(col == head2 + 1, out16, 0.0), axis=1)
    scale = jnp.maximum(jax.nn.softplus(pre + _BIAS), 1e-4)
    loc_ref[...] = loc.reshape(10, 8)
    scale_ref[...] = scale.reshape(10, 8)


def kernel(x_joint, x_torso, Wj, bj, Wt, bt,
           W1_tj_rel, b1_tj, W1_tj_root, W1_jj_rel, b1_jj, W1_jj_root,
           W1_jt_rel, b1_jt, W1_jt_root,
           W2_tj_rel, b2_tj, W2_tj_root, W2_jj_rel, b2_jj, W2_jj_root,
           W2_jt_rel, b2_jt, W2_jt_root,
           Wout, bout, ei_tj, ei_jt, ei_jj):
    args = (x_joint, x_torso, Wj, bj, Wt, bt,
            W1_tj_rel, b1_tj, W1_tj_root, W1_jj_rel, b1_jj, W1_jj_root,
            W1_jt_rel, b1_jt, W1_jt_root,
            W2_tj_rel, b2_tj, W2_tj_root, W2_jj_rel, b2_jj, W2_jj_root,
            W2_jt_rel, b2_jt, W2_jt_root,
            Wout, bout, ei_tj.astype(jnp.int32), ei_jt.astype(jnp.int32),
            ei_jj.astype(jnp.int32))
    scratch = [pltpu.VMEM(*_SHAPES[i]) for i in _LIVE]
    scratch.append(pltpu.SemaphoreType.DMA((len(_LIVE),)))
    loc, scale = pl.pallas_call(
        _body,
        in_specs=[pl.BlockSpec(memory_space=_ANY)] * _N_IN,
        out_shape=(jax.ShapeDtypeStruct((10, 8), _F32),
                   jax.ShapeDtypeStruct((10, 8), _F32)),
        scratch_shapes=scratch,
    )(*args)
    return (loc, scale)


# speedup vs baseline: 2.7477x; 1.1105x over previous
import jax
import jax.numpy as jnp
import numpy as np
from jax.experimental import pallas as pl

_F32 = jnp.float32
_BIAS = float(np.log(np.expm1(1.0)))


def _adj(edge_ref, n_src, n_dst):
    e = edge_ref[...]
    src = e[0, :]
    dst = e[1, :]
    n_e = src.shape[0]
    s_oh = (src[:, None] == jax.lax.broadcasted_iota(jnp.int32, (n_e, n_src), 1)
            ).astype(_F32)
    d_oh = (dst[:, None] == jax.lax.broadcasted_iota(jnp.int32, (n_e, n_dst), 1)
            ).astype(_F32)
    return jax.lax.dot_general(
        d_oh, s_oh, (((0,), (0,)), ((), ())), preferred_element_type=_F32)


def _mm(a, b):
    return jax.lax.dot_general(
        a, b, (((1,), (0,)), ((), ())), preferred_element_type=_F32)


def _body(x_joint, x_torso, Wj, bj, Wt, bt,
          W1_tj_rel, b1_tj, W1_tj_root, W1_jj_rel, b1_jj, W1_jj_root,
          W1_jt_rel, b1_jt, W1_jt_root,
          W2_tj_rel, b2_tj, W2_tj_root, W2_jj_rel, b2_jj, W2_jj_root,
          Wout, bout, ei_tj, ei_jt, ei_jj, loc_ref, scale_ref):
    h_j = _mm(x_joint[...], Wj[...]) + bj[...][None, :]
    h_t = _mm(x_torso[...], Wt[...]) + bt[...][None, :]

    A_tj = _adj(ei_tj, 10, 80)
    A_jt = _adj(ei_jt, 80, 10)
    A_jj = _adj(ei_jj, 80, 80)

    x1j = jnp.concatenate([_mm(A_tj, h_t), _mm(A_jj, h_j), h_j], axis=1)
    w1j = jnp.concatenate(
        [W1_tj_rel[...], W1_jj_rel[...], W1_tj_root[...] + W1_jj_root[...]],
        axis=0)
    j1 = jnp.tanh(_mm(x1j, w1j) + (b1_tj[...] + b1_jj[...])[None, :])
    x1t = jnp.concatenate([_mm(A_jt, h_j), h_t], axis=1)
    w1t = jnp.concatenate([W1_jt_rel[...], W1_jt_root[...]], axis=0)
    t1 = jnp.tanh(_mm(x1t, w1t) + b1_jt[...][None, :])

    x2j = jnp.concatenate([_mm(A_tj, t1), _mm(A_jj, j1), j1], axis=1)
    w2j = jnp.concatenate(
        [W2_tj_rel[...], W2_jj_rel[...], W2_tj_root[...] + W2_jj_root[...]],
        axis=0)
    j2 = jnp.tanh(_mm(x2j, w2j) + (b2_tj[...] + b2_jj[...])[None, :])

    Wo = Wout[...]
    bo = bout[...]
    Wbig = jnp.concatenate([Wo[h] for h in range(8)], axis=1)
    bbig = jnp.concatenate([bo[h] for h in range(8)], axis=0)
    out16 = _mm(j2, Wbig) + bbig[None, :]
    col = jax.lax.broadcasted_iota(jnp.int32, (80, 16), 1)
    head2 = 2 * (jax.lax.broadcasted_iota(jnp.int32, (80, 16), 0) % 8)
    loc = jnp.sum(jnp.where(col == head2, out16, 0.0), axis=1)
    pre = jnp.sum(jnp.where(col == head2 + 1, out16, 0.0), axis=1)
    scale = jnp.maximum(jax.nn.softplus(pre + _BIAS), 1e-4)
    loc_ref[...] = loc.reshape(10, 8)
    scale_ref[...] = scale.reshape(10, 8)


def kernel(x_joint, x_torso, Wj, bj, Wt, bt,
           W1_tj_rel, b1_tj, W1_tj_root, W1_jj_rel, b1_jj, W1_jj_root,
           W1_jt_rel, b1_jt, W1_jt_root,
           W2_tj_rel, b2_tj, W2_tj_root, W2_jj_rel, b2_jj, W2_jj_root,
           W2_jt_rel, b2_jt, W2_jt_root,
           Wout, bout, ei_tj, ei_jt, ei_jj):
    loc, scale = pl.pallas_call(
        _body,
        out_shape=(jax.ShapeDtypeStruct((10, 8), _F32),
                   jax.ShapeDtypeStruct((10, 8), _F32)),
    )(x_joint, x_torso, Wj, bj, Wt, bt,
      W1_tj_rel, b1_tj, W1_tj_root, W1_jj_rel, b1_jj, W1_jj_root,
      W1_jt_rel, b1_jt, W1_jt_root,
      W2_tj_rel, b2_tj, W2_tj_root, W2_jj_rel, b2_jj, W2_jj_root,
      Wout, bout, ei_tj.astype(jnp.int32), ei_jt.astype(jnp.int32),
      ei_jj.astype(jnp.int32))
    return (loc, scale)


# speedup vs baseline: 2.8263x; 1.0286x over previous
import jax
import jax.numpy as jnp
import numpy as np
from jax.experimental import pallas as pl

_F32 = jnp.float32
_BIAS = float(np.log(np.expm1(1.0)))


def _adj(edge_ref, n_src, n_dst):
    e = edge_ref[...]
    src = e[0, :]
    dst = e[1, :]
    n_e = src.shape[0]
    s_oh = (src[:, None] == jax.lax.broadcasted_iota(jnp.int32, (n_e, n_src), 1)
            ).astype(_F32)
    d_oh = (dst[:, None] == jax.lax.broadcasted_iota(jnp.int32, (n_e, n_dst), 1)
            ).astype(_F32)
    return jax.lax.dot_general(
        d_oh, s_oh, (((0,), (0,)), ((), ())), preferred_element_type=_F32)


def _mm(a, b):
    return jax.lax.dot_general(
        a, b, (((1,), (0,)), ((), ())), preferred_element_type=_F32)


def _body(x_joint, x_torso, Wj, bj, Wt, bt,
          W1_tj_rel, b1_tj, W1_tj_root, W1_jj_rel, b1_jj, W1_jj_root,
          W1_jt_rel, b1_jt, W1_jt_root,
          W2_tj_rel, b2_tj, W2_tj_root, W2_jj_rel, b2_jj, W2_jj_root,
          Wbig, bbig, ei_tj, ei_jt, ei_jj, loc_ref, scale_ref):
    h_j = _mm(x_joint[...], Wj[...]) + bj[...][None, :]
    h_t = _mm(x_torso[...], Wt[...]) + bt[...][None, :]

    A_tj = _adj(ei_tj, 10, 80)
    A_jt = _adj(ei_jt, 80, 10)
    A_jj = _adj(ei_jj, 80, 80)

    x1j = jnp.concatenate([_mm(A_tj, h_t), _mm(A_jj, h_j), h_j], axis=1)
    w1j = jnp.concatenate(
        [W1_tj_rel[...], W1_jj_rel[...], W1_tj_root[...] + W1_jj_root[...]],
        axis=0)
    j1 = jnp.tanh(_mm(x1j, w1j) + (b1_tj[...] + b1_jj[...])[None, :])
    x1t = jnp.concatenate([_mm(A_jt, h_j), h_t], axis=1)
    w1t = jnp.concatenate([W1_jt_rel[...], W1_jt_root[...]], axis=0)
    t1 = jnp.tanh(_mm(x1t, w1t) + b1_jt[...][None, :])

    x2j = jnp.concatenate([_mm(A_tj, t1), _mm(A_jj, j1), j1], axis=1)
    w2j = jnp.concatenate(
        [W2_tj_rel[...], W2_jj_rel[...], W2_tj_root[...] + W2_jj_root[...]],
        axis=0)
    j2 = jnp.tanh(_mm(x2j, w2j) + (b2_tj[...] + b2_jj[...])[None, :])

    out16 = _mm(j2, Wbig[...]) + bbig[...][None, :]
    col = jax.lax.broadcasted_iota(jnp.int32, (80, 16), 1)
    head2 = 2 * (jax.lax.broadcasted_iota(jnp.int32, (80, 16), 0) % 8)
    loc = jnp.sum(jnp.where(col == head2, out16, 0.0), axis=1)
    pre = jnp.sum(jnp.where(col == head2 + 1, out16, 0.0), axis=1)
    scale = jnp.maximum(jax.nn.softplus(pre + _BIAS), 1e-4)
    loc_ref[...] = loc.reshape(10, 8)
    scale_ref[...] = scale.reshape(10, 8)


def kernel(x_joint, x_torso, Wj, bj, Wt, bt,
           W1_tj_rel, b1_tj, W1_tj_root, W1_jj_rel, b1_jj, W1_jj_root,
           W1_jt_rel, b1_jt, W1_jt_root,
           W2_tj_rel, b2_tj, W2_tj_root, W2_jj_rel, b2_jj, W2_jj_root,
           W2_jt_rel, b2_jt, W2_jt_root,
           Wout, bout, ei_tj, ei_jt, ei_jj):
    Wbig = jnp.transpose(Wout, (1, 0, 2)).reshape(64, 16)
    bbig = bout.reshape(16)
    loc, scale = pl.pallas_call(
        _body,
        out_shape=(jax.ShapeDtypeStruct((10, 8), _F32),
                   jax.ShapeDtypeStruct((10, 8), _F32)),
    )(x_joint, x_torso, Wj, bj, Wt, bt,
      W1_tj_rel, b1_tj, W1_tj_root, W1_jj_rel, b1_jj, W1_jj_root,
      W1_jt_rel, b1_jt, W1_jt_root,
      W2_tj_rel, b2_tj, W2_tj_root, W2_jj_rel, b2_jj, W2_jj_root,
      Wbig, bbig, ei_tj.astype(jnp.int32), ei_jt.astype(jnp.int32),
      ei_jj.astype(jnp.int32))
    return (loc, scale)
